# baseline (speedup 1.0000x reference)
import jax
import jax.numpy as jnp
from jax import lax
from jax.experimental import pallas as pl
from jax.experimental.pallas import tpu as pltpu
from jax.experimental.pallas import tpu_sc as plsc

NUM_TYPES = 10
D = 128
B_TOTAL = 16384 * 200
NC, NS = 2, 16
NW = NC * NS
B_PER_W = B_TOTAL // NW
K = 2
C = K * 128
N_CHUNKS = B_PER_W // C
N_PAIRS = N_CHUNKS // 2

EXT_BYTES = 128 * 4
ROW_BYTES = D * 4


def _embed_kernel(
    x_hbm, table_hbm, out_hbm,
    pat0, pat1, xf0, xf1, idx0, idx1, rows0, rows1,
    sem_x0, sem_x1, sem_g0, sem_g1, sem_o0, sem_o1,
):
    wid = lax.axis_index("s") * NC + lax.axis_index("c")
    base = wid * B_PER_W
    lane = jnp.arange(16, dtype=jnp.int32)

    def start_ext(i, pat, xf, sem):
        off = (base + i * C) * 8
        for r in range(C // 16):
            pat[pl.ds(r * 16, 16)] = (r * 16 + lane) * 8 + 7 + off
        for j in range(K):
            pltpu.make_async_copy(
                x_hbm.at[pat.at[pl.ds(j * 128, 128)]],
                xf.at[pl.ds(j * 128, 128)],
                sem,
            ).start()

    def finish_ext_start_gather(pat, xf, idx, rows, sem, sem_g):
        for j in range(K):
            pltpu.make_async_copy(
                x_hbm.at[pat.at[pl.ds(j * 128, 128)]],
                xf.at[pl.ds(j * 128, 128)],
                sem,
            ).wait()
        for r in range(C // 16):
            idx[pl.ds(r * 16, 16)] = xf[pl.ds(r * 16, 16)].astype(jnp.int32)
        for j in range(K):
            pltpu.make_async_copy(
                table_hbm.at[idx.at[pl.ds(j * 128, 128)]],
                rows.at[pl.ds(j * 128, 128)],
                sem_g,
            ).start()

    def finish_gather_start_write(i, idx, rows, sem_g, sem_o):
        for j in range(K):
            pltpu.make_async_copy(
                table_hbm.at[idx.at[pl.ds(j * 128, 128)]],
                rows.at[pl.ds(j * 128, 128)],
                sem_g,
            ).wait()
        pltpu.make_async_copy(
            rows, out_hbm.at[pl.ds(base + i * C, C)], sem_o
        ).start()

    def wait_write(i, rows, sem_o):
        pltpu.make_async_copy(
            rows, out_hbm.at[pl.ds(base + i * C, C)], sem_o
        ).wait()

    start_ext(0, pat0, xf0, sem_x0)

    def body(t, carry):
        i0 = 2 * t
        i1 = i0 + 1
        start_ext(i1, pat1, xf1, sem_x1)

        @pl.when(t > 0)
        def _():
            wait_write(i0 - 2, rows0, sem_o0)

        finish_ext_start_gather(pat0, xf0, idx0, rows0, sem_x0, sem_g0)

        @pl.when(t < N_PAIRS - 1)
        def _():
            start_ext(i0 + 2, pat0, xf0, sem_x0)

        @pl.when(t > 0)
        def _():
            wait_write(i1 - 2, rows1, sem_o1)

        finish_ext_start_gather(pat1, xf1, idx1, rows1, sem_x1, sem_g1)
        finish_gather_start_write(i0, idx0, rows0, sem_g0, sem_o0)
        finish_gather_start_write(i1, idx1, rows1, sem_g1, sem_o1)
        return carry

    lax.fori_loop(0, N_PAIRS, body, 0)
    wait_write(N_CHUNKS - 2, rows0, sem_o0)
    wait_write(N_CHUNKS - 1, rows1, sem_o1)


@jax.jit
def kernel(x, table):
    x2 = x.reshape(B_TOTAL * 8)
    mesh = plsc.VectorSubcoreMesh(core_axis_name="c", subcore_axis_name="s")
    out = pl.kernel(
        _embed_kernel,
        mesh=mesh,
        out_type=jax.ShapeDtypeStruct((B_TOTAL, D), jnp.float32),
        scratch_types=[
            pltpu.VMEM((C,), jnp.int32),
            pltpu.VMEM((C,), jnp.int32),
            pltpu.VMEM((C,), jnp.float32),
            pltpu.VMEM((C,), jnp.float32),
            pltpu.VMEM((C,), jnp.int32),
            pltpu.VMEM((C,), jnp.int32),
            pltpu.VMEM((C, D), jnp.float32),
            pltpu.VMEM((C, D), jnp.float32),
            pltpu.SemaphoreType.DMA,
            pltpu.SemaphoreType.DMA,
            pltpu.SemaphoreType.DMA,
            pltpu.SemaphoreType.DMA,
            pltpu.SemaphoreType.DMA,
            pltpu.SemaphoreType.DMA,
        ],
    )(x2, table)
    return out.reshape(16384, 200, D)


# speedup vs baseline: 3.4759x; 3.4759x over previous
import jax
import jax.numpy as jnp
from jax import lax
from jax.experimental import pallas as pl
from jax.experimental.pallas import tpu as pltpu
from jax.experimental.pallas import tpu_sc as plsc

NUM_TYPES = 10
D = 128
B_TOTAL = 16384 * 200
NC, NS = 2, 16
NW = NC * NS
B_PER_W = B_TOTAL // NW
C = 256
N_CHUNKS = B_PER_W // C
N_PAIRS = N_CHUNKS // 2
U = 8


def _embed_kernel(
    x_hbm, table_hbm, out_hbm,
    tab_v, x0, x1, r0, r1,
    sem_x0, sem_x1, sem_o0, sem_o1,
):
    wid = lax.axis_index("s") * NC + lax.axis_index("c")
    base = wid * B_PER_W

    pltpu.sync_copy(table_hbm, tab_v)

    def start_x(i, xv, sem):
        pltpu.make_async_copy(
            x_hbm.at[pl.ds((base + i * C) * 8, C * 8)], xv, sem
        ).start()

    def wait_x(i, xv, sem):
        pltpu.make_async_copy(
            x_hbm.at[pl.ds((base + i * C) * 8, C * 8)], xv, sem
        ).wait()

    def start_write(i, rv, sem):
        pltpu.make_async_copy(
            rv, out_hbm.at[pl.ds((base + i * C) * D, C * D)], sem
        ).start()

    def wait_write(i, rv, sem):
        pltpu.make_async_copy(
            rv, out_hbm.at[pl.ds((base + i * C) * D, C * D)], sem
        ).wait()

    def expand(xv, rv):
        def erow(rr, carry):
            for p in range(U // 2):
                r = rr * U + 2 * p
                v = xv[pl.ds(r * 8, 16)]
                t0 = v[7].astype(jnp.int32) * D
                t1 = v[15].astype(jnp.int32) * D
                for c in range(D // 16):
                    rv[pl.ds(r * D + c * 16, 16)] = tab_v[pl.ds(t0 + c * 16, 16)]
                for c in range(D // 16):
                    rv[pl.ds((r + 1) * D + c * 16, 16)] = tab_v[pl.ds(t1 + c * 16, 16)]
            return carry
        lax.fori_loop(0, C // U, erow, 0)

    start_x(0, x0, sem_x0)
    start_x(1, x1, sem_x1)

    def body(t, carry):
        i0 = 2 * t
        i1 = i0 + 1

        wait_x(i0, x0, sem_x0)

        @pl.when(t > 0)
        def _():
            wait_write(i0 - 2, r0, sem_o0)

        expand(x0, r0)

        @pl.when(t < N_PAIRS - 1)
        def _():
            start_x(i0 + 2, x0, sem_x0)

        start_write(i0, r0, sem_o0)

        wait_x(i1, x1, sem_x1)

        @pl.when(t > 0)
        def _():
            wait_write(i1 - 2, r1, sem_o1)

        expand(x1, r1)

        @pl.when(t < N_PAIRS - 1)
        def _():
            start_x(i1 + 2, x1, sem_x1)

        start_write(i1, r1, sem_o1)
        return carry

    lax.fori_loop(0, N_PAIRS, body, 0)
    wait_write(N_CHUNKS - 2, r0, sem_o0)
    wait_write(N_CHUNKS - 1, r1, sem_o1)


@jax.jit
def kernel(x, table):
    x2 = x.reshape(B_TOTAL * 8)
    tab = table.reshape(NUM_TYPES * D)
    mesh = plsc.VectorSubcoreMesh(core_axis_name="c", subcore_axis_name="s")
    out = pl.kernel(
        _embed_kernel,
        mesh=mesh,
        out_type=jax.ShapeDtypeStruct((B_TOTAL * D,), jnp.float32),
        scratch_types=[
            pltpu.VMEM((NUM_TYPES * D,), jnp.float32),
            pltpu.VMEM((C * 8,), jnp.float32),
            pltpu.VMEM((C * 8,), jnp.float32),
            pltpu.VMEM((C * D,), jnp.float32),
            pltpu.VMEM((C * D,), jnp.float32),
            pltpu.SemaphoreType.DMA,
            pltpu.SemaphoreType.DMA,
            pltpu.SemaphoreType.DMA,
            pltpu.SemaphoreType.DMA,
        ],
    )(x2, tab)
    return out.reshape(16384, 200, D)


# speedup vs baseline: 7.4477x; 2.1427x over previous
import jax
import jax.numpy as jnp
from jax import lax
from jax.experimental import pallas as pl
from jax.experimental.pallas import tpu as pltpu
from jax.experimental.pallas import tpu_sc as plsc

NUM_TYPES = 10
D = 128
B_TOTAL = 16384 * 200
NC, NS = 2, 16
NW = NC * NS
B_PER_W = B_TOTAL // NW
C = 256
N_CHUNKS = B_PER_W // C
N_PAIRS = N_CHUNKS // 2
U = 16


def _embed_kernel(
    x_hbm, table_hbm, out_hbm,
    tab_v, x0, x1, r0, r1,
    sem_x0, sem_x1, sem_o0, sem_o1,
):
    wid = lax.axis_index("s") * NC + lax.axis_index("c")
    base = wid * B_PER_W

    pltpu.sync_copy(table_hbm, tab_v)

    def start_x(i, xv, sem):
        pltpu.make_async_copy(
            x_hbm.at[pl.ds((base + i * C) * 8, C * 8)], xv, sem
        ).start()

    def wait_x(i, xv, sem):
        pltpu.make_async_copy(
            x_hbm.at[pl.ds((base + i * C) * 8, C * 8)], xv, sem
        ).wait()

    def start_write(i, rv, sem):
        pltpu.make_async_copy(
            rv, out_hbm.at[pl.ds((base + i * C) * D, C * D)], sem
        ).start()

    def wait_write(i, rv, sem):
        pltpu.make_async_copy(
            rv, out_hbm.at[pl.ds((base + i * C) * D, C * D)], sem
        ).wait()

    def expand(xv, rv):
        def erow(rr, carry):
            r0_ = rr * U
            offs = []
            for p in range(U // 2):
                v = xv[pl.ds((r0_ + 2 * p) * 8, 16)]
                offs.append(v[7].astype(jnp.int32) * D)
                offs.append(v[15].astype(jnp.int32) * D)
            for u in range(U):
                vals = [
                    tab_v[pl.ds(offs[u] + c * 16, 16)] for c in range(D // 16)
                ]
                for c in range(D // 16):
                    rv[pl.ds((r0_ + u) * D + c * 16, 16)] = vals[c]
            return carry
        lax.fori_loop(0, C // U, erow, 0)

    start_x(0, x0, sem_x0)
    start_x(1, x1, sem_x1)

    def body(t, carry):
        i0 = 2 * t
        i1 = i0 + 1

        wait_x(i0, x0, sem_x0)

        @pl.when(t > 0)
        def _():
            wait_write(i0 - 2, r0, sem_o0)

        expand(x0, r0)

        @pl.when(t < N_PAIRS - 1)
        def _():
            start_x(i0 + 2, x0, sem_x0)

        start_write(i0, r0, sem_o0)

        wait_x(i1, x1, sem_x1)

        @pl.when(t > 0)
        def _():
            wait_write(i1 - 2, r1, sem_o1)

        expand(x1, r1)

        @pl.when(t < N_PAIRS - 1)
        def _():
            start_x(i1 + 2, x1, sem_x1)

        start_write(i1, r1, sem_o1)
        return carry

    lax.fori_loop(0, N_PAIRS, body, 0)
    wait_write(N_CHUNKS - 2, r0, sem_o0)
    wait_write(N_CHUNKS - 1, r1, sem_o1)


@jax.jit
def kernel(x, table):
    x2 = x.reshape(B_TOTAL * 8)
    tab = table.reshape(NUM_TYPES * D)
    mesh = plsc.VectorSubcoreMesh(core_axis_name="c", subcore_axis_name="s")
    out = pl.kernel(
        _embed_kernel,
        mesh=mesh,
        out_type=jax.ShapeDtypeStruct((B_TOTAL * D,), jnp.float32),
        scratch_types=[
            pltpu.VMEM((NUM_TYPES * D,), jnp.float32),
            pltpu.VMEM((C * 8,), jnp.float32),
            pltpu.VMEM((C * 8,), jnp.float32),
            pltpu.VMEM((C * D,), jnp.float32),
            pltpu.VMEM((C * D,), jnp.float32),
            pltpu.SemaphoreType.DMA,
            pltpu.SemaphoreType.DMA,
            pltpu.SemaphoreType.DMA,
            pltpu.SemaphoreType.DMA,
        ],
    )(x2, tab)
    return out.reshape(16384, 200, D)
